# baseline (speedup 1.0000x reference)
import functools

import jax
import jax.numpy as jnp
from jax import lax
from jax.experimental import pallas as pl
from jax.experimental.pallas import tpu as pltpu
from jax.experimental.pallas import tpu_sc as plsc

N = 10000
E = 160000
D_IN = 256
D_H = 512
DHALF = 128

NC = 2
NS = 16
CH = 128

NP = 10240
EP = 172032
N_TRASH = 64

BLK = 256
_MESH = plsc.VectorSubcoreMesh(core_axis_name="c", subcore_axis_name="s")


_CPT1 = EP // CH // (NC * NS)


@functools.partial(
    pl.kernel,
    out_type=jax.ShapeDtypeStruct((NC, NP), jnp.float32),
    mesh=_MESH,
    scratch_types=[
        pltpu.VMEM((_CPT1, 1, CH), jnp.int32),
        pltpu.VMEM((CH,), jnp.float32),
        pltpu.VMEM((NP // NS,), jnp.float32),
        pltpu.VMEM_SHARED((NP,), jnp.float32),
        pltpu.SemaphoreType.DMA,
    ],
)
def _deg_kernel(dst2d_hbm, out_hbm, didx, ones_v, stage_v, acc_sh, sem):
    c = lax.axis_index("c")
    s = lax.axis_index("s")
    rows = NP // NS
    base = s * rows

    def _fill(i, _):
        ones_v[pl.ds(i * 16, 16)] = jnp.ones((16,), jnp.float32)
        return ()
    lax.fori_loop(0, CH // 16, _fill, ())

    def _zero(i, _):
        stage_v[pl.ds(i * 16, 16)] = jnp.zeros((16,), jnp.float32)
        return ()
    lax.fori_loop(0, rows // 16, _zero, ())
    pltpu.sync_copy(stage_v, acc_sh.at[pl.ds(base, rows)])
    chunk0 = (c * NS + s) * _CPT1
    pltpu.sync_copy(dst2d_hbm.at[pl.ds(chunk0, _CPT1)], didx)
    plsc.subcore_barrier()

    def _body(p, _):
        for b in range(6):
            pltpu.async_copy(ones_v, acc_sh.at[didx.at[p * 6 + b, 0]], sem,
                             add=True)
        for b in range(6):
            pltpu.make_async_copy(ones_v, acc_sh.at[didx.at[0, 0]],
                                  sem).wait()
        return ()
    lax.fori_loop(0, _CPT1 // 6, _body, ())
    plsc.subcore_barrier()

    pltpu.sync_copy(acc_sh.at[pl.ds(base, rows)], stage_v)
    pltpu.sync_copy(stage_v, out_hbm.at[c, pl.ds(base, rows)])


_PBLK = 1024


def _prescale_body(p0_ref, p1_ref, x_ref, dinv_ref, xs0_ref, xs1_ref):
    deg = p0_ref[...] + p1_ref[...]
    dinv = lax.rsqrt(jnp.maximum(deg, 1.0))
    dinv_ref[...] = dinv
    xs = x_ref[...] * dinv[:, None]
    xs0_ref[...] = xs[:, :DHALF]
    xs1_ref[...] = xs[:, DHALF:]


def _prescale(deg_flat, x):
    nblk = NP // _PBLK
    return pl.pallas_call(
        _prescale_body,
        grid=(nblk,),
        in_specs=[
            pl.BlockSpec((_PBLK,), lambda i: (i,)),
            pl.BlockSpec((_PBLK,), lambda i: (i + NP // _PBLK,)),
            pl.BlockSpec((_PBLK, D_IN), lambda i: (i, 0)),
        ],
        out_specs=[
            pl.BlockSpec((_PBLK,), lambda i: (i,)),
            pl.BlockSpec((_PBLK, DHALF), lambda i: (i, 0)),
            pl.BlockSpec((_PBLK, DHALF), lambda i: (i, 0)),
        ],
        out_shape=[
            jax.ShapeDtypeStruct((NP,), jnp.float32),
            jax.ShapeDtypeStruct((NP, DHALF), jnp.float32),
            jax.ShapeDtypeStruct((NP, DHALF), jnp.float32),
        ],
    )(deg_flat, deg_flat, x)


CH3 = 64
_CPT3 = EP // CH3 // NS


_GRP = 6
_NGRP = _CPT3 // _GRP


@functools.partial(
    pl.kernel,
    out_type=jax.ShapeDtypeStruct((NC, NP, DHALF), jnp.float32),
    mesh=_MESH,
    scratch_types=[
        pltpu.VMEM((_GRP, 1, CH3), jnp.int32),
        pltpu.VMEM((_GRP, 1, CH3), jnp.int32),
        pltpu.VMEM((_GRP, 1, CH3), jnp.int32),
        pltpu.VMEM((_GRP, 1, CH3), jnp.int32),
        pltpu.VMEM((CH3, DHALF), jnp.float32),
        pltpu.VMEM((CH3, DHALF), jnp.float32),
        pltpu.VMEM((CH3, DHALF), jnp.float32),
        pltpu.VMEM((16, DHALF), jnp.float32),
        pltpu.VMEM_SHARED((NP, DHALF), jnp.float32),
        pltpu.SemaphoreType.DMA,
        pltpu.SemaphoreType.DMA,
        pltpu.SemaphoreType.DMA,
        pltpu.SemaphoreType.DMA,
        pltpu.SemaphoreType.DMA,
        pltpu.SemaphoreType.DMA,
        pltpu.SemaphoreType.DMA,
    ],
)
def _agg_kernel(xs0, xs1, src2d, dst2d, out_hbm,
                gsa, gda, gsb, gdb, rb0, rb1, rb2, zblk, acc_sh,
                g0, g1, g2, s0, s1, s2, isem):
    c = lax.axis_index("c")
    s = lax.axis_index("s")
    rows = NP // NS
    base = s * rows
    rbs = (rb0, rb1, rb2)
    gsem = (g0, g1, g2)
    ssem = (s0, s1, s2)

    def _zfill(i, _):
        for j in range(DHALF // 16):
            zblk[i, pl.ds(j * 16, 16)] = jnp.zeros((16,), jnp.float32)
        return ()
    lax.fori_loop(0, 16, _zfill, ())

    def _zero(i, _):
        pltpu.sync_copy(zblk, acc_sh.at[pl.ds(base + i * 16, 16), :])
        return ()
    lax.fori_loop(0, rows // 16, _zero, ())
    plsc.subcore_barrier()

    chunk0 = s * _CPT3

    def _run(xs_hbm):
        def g_start(idx_row, b):
            pltpu.async_copy(xs_hbm.at[idx_row], rbs[b], gsem[b])

        def g_wait(b):
            pltpu.make_async_copy(xs_hbm.at[gsa.at[0, 0]], rbs[b],
                                  gsem[b]).wait()

        def s_start(idx_row, b):
            pltpu.async_copy(rbs[b], acc_sh.at[idx_row], ssem[b], add=True)

        def s_wait(b):
            pltpu.make_async_copy(rbs[b], acc_sh.at[gda.at[0, 0]],
                                  ssem[b]).wait()

        def i_wait():
            pltpu.make_async_copy(src2d.at[pl.ds(0, _GRP)], gsa,
                                  isem).wait()

        def do_group(g, cur_src, cur_dst, nxt_src, nxt_dst, first):
            for kk in range(_GRP):
                b = kk % 3
                if not (first and kk < 3):
                    s_wait(b)
                g_start(cur_src.at[kk, 0], b)
                if not (first and kk == 0):
                    pb = (kk - 1) % 3
                    g_wait(pb)
                    if kk == 0:
                        s_start(nxt_dst.at[_GRP - 1, 0], pb)
                    else:
                        s_start(cur_dst.at[kk - 1, 0], pb)
                if kk == 2:
                    start = chunk0 + jnp.minimum((g + 1) * _GRP,
                                                 _CPT3 - _GRP)
                    pltpu.async_copy(src2d.at[pl.ds(start, _GRP)],
                                     nxt_src, isem)
                    pltpu.async_copy(dst2d.at[pl.ds(start, _GRP)],
                                     nxt_dst, isem)
                if kk == _GRP - 1:
                    i_wait()
                    i_wait()

        pltpu.sync_copy(src2d.at[pl.ds(chunk0, _GRP)], gsa)
        pltpu.sync_copy(dst2d.at[pl.ds(chunk0, _GRP)], gda)
        do_group(0, gsa, gda, gsb, gdb, True)

        def _body(p, _):
            g = 2 * p + 1
            do_group(g, gsb, gdb, gsa, gda, False)
            do_group(g + 1, gsa, gda, gsb, gdb, False)
            return ()
        lax.fori_loop(0, (_NGRP - 2) // 2, _body, ())
        do_group(_NGRP - 1, gsb, gdb, gsa, gda, False)

        lastb = (_CPT3 - 1) % 3
        g_wait(lastb)
        s_wait((_CPT3 - 3) % 3)
        s_wait((_CPT3 - 2) % 3)
        s_start(gdb.at[_GRP - 1, 0], lastb)
        s_wait(lastb)

    @pl.when(c == 0)
    def _():
        _run(xs0)

    @pl.when(c == 1)
    def _():
        _run(xs1)

    plsc.subcore_barrier()

    def _out(i, _):
        r0 = base + i * CH3
        pltpu.sync_copy(acc_sh.at[pl.ds(r0, CH3), :], rb0)
        pltpu.sync_copy(rb0, out_hbm.at[c, pl.ds(r0, CH3), :])
        return ()
    lax.fori_loop(0, rows // CH3, _out, ())


_DBLK = 512


def _dense_body(dinv_ref, a0_ref, a1_ref, w1_ref, b1_ref, w2_ref, t_ref):
    dinv = dinv_ref[...]
    z0 = (a0_ref[0] * dinv[:, None]).astype(jnp.bfloat16)
    z1 = (a1_ref[0] * dinv[:, None]).astype(jnp.bfloat16)
    h = jnp.dot(z0, w1_ref[0:DHALF, :], preferred_element_type=jnp.float32)
    h += jnp.dot(z1, w1_ref[DHALF:D_IN, :], preferred_element_type=jnp.float32)
    h = jnp.maximum(h + b1_ref[...][None, :], 0.0).astype(jnp.bfloat16)
    s = jnp.dot(h, w2_ref[...], preferred_element_type=jnp.float32)
    t_ref[...] = s[:, 0] * dinv


def _dense(dinv, acc, W1, b1, w2pad):
    return pl.pallas_call(
        _dense_body,
        grid=(NP // _DBLK,),
        in_specs=[
            pl.BlockSpec((_DBLK,), lambda i: (i,)),
            pl.BlockSpec((1, _DBLK, DHALF), lambda i: (0, i, 0)),
            pl.BlockSpec((1, _DBLK, DHALF), lambda i: (1, i, 0)),
            pl.BlockSpec((D_IN, D_H), lambda i: (0, 0)),
            pl.BlockSpec((D_H,), lambda i: (0,)),
            pl.BlockSpec((D_H, 128), lambda i: (0, 0)),
        ],
        out_specs=pl.BlockSpec((_DBLK,), lambda i: (i,)),
        out_shape=jax.ShapeDtypeStruct((NP,), jnp.float32),
    )(dinv, acc, acc, W1, b1, w2pad)


_CPT5 = EP // CH // (NC * NS)


@functools.partial(
    pl.kernel,
    out_type=jax.ShapeDtypeStruct((NC, NP), jnp.float32),
    mesh=_MESH,
    compiler_params=pltpu.CompilerParams(needs_layout_passes=False),
    scratch_types=[
        pltpu.VMEM((_CPT5, 1, CH), jnp.int32),
        pltpu.VMEM((_CPT5, 1, CH), jnp.int32),
        pltpu.VMEM((NP,), jnp.float32),
        pltpu.VMEM((CH,), jnp.float32),
        pltpu.VMEM((CH,), jnp.float32),
        pltpu.VMEM((NP // NS,), jnp.float32),
        pltpu.VMEM_SHARED((NP,), jnp.float32),
        pltpu.SemaphoreType.DMA,
        pltpu.SemaphoreType.DMA,
    ],
)
def _agg2_kernel(t_hbm, src2d, dst2d, out_hbm,
                 gidx, sidx, tloc, msga, msgb, stage_v, acc_sh, sema, semb):
    c = lax.axis_index("c")
    s = lax.axis_index("s")
    rows = NP // NS
    base = s * rows

    def _zero(i, _):
        stage_v[pl.ds(i * 16, 16)] = jnp.zeros((16,), jnp.float32)
        return ()
    lax.fori_loop(0, rows // 16, _zero, ())
    pltpu.sync_copy(stage_v, acc_sh.at[pl.ds(base, rows)])
    chunk0 = (c * NS + s) * _CPT5
    pltpu.sync_copy(src2d.at[pl.ds(chunk0, _CPT5)], gidx)
    pltpu.sync_copy(dst2d.at[pl.ds(chunk0, _CPT5)], sidx)
    pltpu.sync_copy(t_hbm, tloc)
    plsc.subcore_barrier()

    def _msgs(j, buf):
        for v in range(CH // 16):
            idx = gidx[j, 0, pl.ds(v * 16, 16)]
            buf[pl.ds(v * 16, 16)] = plsc.load_gather(tloc, [idx])

    def _s_start(j, buf, sem):
        pltpu.async_copy(buf, acc_sh.at[sidx.at[j, 0]], sem, add=True)

    def _s_wait(buf, sem):
        pltpu.make_async_copy(buf, acc_sh.at[sidx.at[0, 0]], sem).wait()

    _msgs(0, msga)
    _s_start(0, msga, sema)
    _msgs(1, msgb)
    _s_start(1, msgb, semb)

    def _body(p, _):
        j = p * 2
        _s_wait(msga, sema)
        _msgs(j, msga)
        _s_start(j, msga, sema)
        _s_wait(msgb, semb)
        _msgs(j + 1, msgb)
        _s_start(j + 1, msgb, semb)
        return ()
    lax.fori_loop(1, _CPT5 // 2, _body, ())
    _s_wait(msga, sema)
    _s_wait(msgb, semb)
    plsc.subcore_barrier()

    pltpu.sync_copy(acc_sh.at[pl.ds(base, rows)], stage_v)
    pltpu.sync_copy(stage_v, out_hbm.at[c, pl.ds(base, rows)])


def _final_body(part_ref, dinv_ref, b2_ref, out_ref):
    v = (part_ref[0, :] + part_ref[1, :]) * dinv_ref[...] + b2_ref[0]
    out_ref[...] = lax.slice(jax.nn.sigmoid(v), (0,), (N,))


def _finalize(part, dinv, b2):
    return pl.pallas_call(
        _final_body,
        in_specs=[
            pl.BlockSpec((NC, NP), lambda: (0, 0)),
            pl.BlockSpec((NP,), lambda: (0,)),
            pl.BlockSpec(memory_space=pltpu.SMEM),
        ],
        out_specs=pl.BlockSpec((N,), lambda: (0,)),
        out_shape=jax.ShapeDtypeStruct((N,), jnp.float32),
    )(part, dinv, b2)


def kernel(x, edge_index, W1, b1, W2, b2):
    pad_e = EP - E - N
    loop = jnp.arange(N, dtype=jnp.int32)
    trash = (N + (jnp.arange(pad_e, dtype=jnp.int32) % N_TRASH))
    src_flat = jnp.concatenate([edge_index[0].astype(jnp.int32), loop, trash])
    dst_flat = jnp.concatenate([edge_index[1].astype(jnp.int32), loop, trash])
    src2d = src_flat.reshape(EP // CH, 1, CH)
    dst2d = dst_flat.reshape(EP // CH, 1, CH)
    deg2 = _deg_kernel(dst2d)
    dinv, xs0, xs1 = _prescale(deg2.reshape(NC * NP), x)
    acc = _agg_kernel(xs0, xs1, src_flat.reshape(EP // CH3, 1, CH3),
                      dst_flat.reshape(EP // CH3, 1, CH3))
    w2pad = jnp.pad(W2.astype(jnp.bfloat16), ((0, 0), (0, 127)))
    t = _dense(dinv, acc, W1.astype(jnp.bfloat16), b1, w2pad)
    part = _agg2_kernel(t, src2d, dst2d)
    return _finalize(part, dinv, b2)[:, None]


# speedup vs baseline: 1.0072x; 1.0072x over previous
import functools

import jax
import jax.numpy as jnp
from jax import lax
from jax.experimental import pallas as pl
from jax.experimental.pallas import tpu as pltpu
from jax.experimental.pallas import tpu_sc as plsc

N = 10000
E = 160000
D_IN = 256
D_H = 512
DHALF = 128

NC = 2
NS = 16
CH = 128

NP = 10240
EP = 172032
N_TRASH = 64

BLK = 256
_MESH = plsc.VectorSubcoreMesh(core_axis_name="c", subcore_axis_name="s")


_CPT1 = EP // CH // (NC * NS)


@functools.partial(
    pl.kernel,
    out_type=jax.ShapeDtypeStruct((NC, NP), jnp.float32),
    mesh=_MESH,
    scratch_types=[
        pltpu.VMEM((_CPT1, 1, CH), jnp.int32),
        pltpu.VMEM((CH,), jnp.float32),
        pltpu.VMEM((NP // NS,), jnp.float32),
        pltpu.VMEM_SHARED((NP,), jnp.float32),
        pltpu.SemaphoreType.DMA,
    ],
)
def _deg_kernel(dst2d_hbm, out_hbm, didx, ones_v, stage_v, acc_sh, sem):
    c = lax.axis_index("c")
    s = lax.axis_index("s")
    rows = NP // NS
    base = s * rows

    def _fill(i, _):
        ones_v[pl.ds(i * 16, 16)] = jnp.ones((16,), jnp.float32)
        return ()
    lax.fori_loop(0, CH // 16, _fill, ())

    def _zero(i, _):
        stage_v[pl.ds(i * 16, 16)] = jnp.zeros((16,), jnp.float32)
        return ()
    lax.fori_loop(0, rows // 16, _zero, ())
    pltpu.sync_copy(stage_v, acc_sh.at[pl.ds(base, rows)])
    chunk0 = (c * NS + s) * _CPT1
    pltpu.sync_copy(dst2d_hbm.at[pl.ds(chunk0, _CPT1)], didx)
    plsc.subcore_barrier()

    def _body(p, _):
        for b in range(6):
            pltpu.async_copy(ones_v, acc_sh.at[didx.at[p * 6 + b, 0]], sem,
                             add=True)
        for b in range(6):
            pltpu.make_async_copy(ones_v, acc_sh.at[didx.at[0, 0]],
                                  sem).wait()
        return ()
    lax.fori_loop(0, _CPT1 // 6, _body, ())
    plsc.subcore_barrier()

    pltpu.sync_copy(acc_sh.at[pl.ds(base, rows)], stage_v)
    pltpu.sync_copy(stage_v, out_hbm.at[c, pl.ds(base, rows)])


_PBLK = 1024


def _prescale_body(p0_ref, p1_ref, x_ref, dinv_ref, xs0_ref, xs1_ref):
    deg = p0_ref[...] + p1_ref[...]
    dinv = lax.rsqrt(jnp.maximum(deg, 1.0))
    dinv_ref[...] = dinv
    xs = x_ref[...] * dinv[:, None]
    xs0_ref[...] = xs[:, :DHALF]
    xs1_ref[...] = xs[:, DHALF:]


def _prescale(deg_flat, x):
    nblk = NP // _PBLK
    return pl.pallas_call(
        _prescale_body,
        grid=(nblk,),
        in_specs=[
            pl.BlockSpec((_PBLK,), lambda i: (i,)),
            pl.BlockSpec((_PBLK,), lambda i: (i + NP // _PBLK,)),
            pl.BlockSpec((_PBLK, D_IN), lambda i: (i, 0)),
        ],
        out_specs=[
            pl.BlockSpec((_PBLK,), lambda i: (i,)),
            pl.BlockSpec((_PBLK, DHALF), lambda i: (i, 0)),
            pl.BlockSpec((_PBLK, DHALF), lambda i: (i, 0)),
        ],
        out_shape=[
            jax.ShapeDtypeStruct((NP,), jnp.float32),
            jax.ShapeDtypeStruct((NP, DHALF), jnp.float32),
            jax.ShapeDtypeStruct((NP, DHALF), jnp.float32),
        ],
    )(deg_flat, deg_flat, x)


_CPT3 = EP // CH // NS


_GRP = 6
_NGRP = _CPT3 // _GRP


@functools.partial(
    pl.kernel,
    out_type=jax.ShapeDtypeStruct((NC, NP, DHALF), jnp.float32),
    mesh=_MESH,
    scratch_types=[
        pltpu.VMEM((_GRP, 1, CH), jnp.int32),
        pltpu.VMEM((_GRP, 1, CH), jnp.int32),
        pltpu.VMEM((_GRP, 1, CH), jnp.int32),
        pltpu.VMEM((_GRP, 1, CH), jnp.int32),
        pltpu.VMEM((CH, DHALF), jnp.float32),
        pltpu.VMEM((CH, DHALF), jnp.float32),
        pltpu.VMEM((16, DHALF), jnp.float32),
        pltpu.VMEM_SHARED((NP, DHALF), jnp.float32),
        pltpu.SemaphoreType.DMA,
        pltpu.SemaphoreType.DMA,
        pltpu.SemaphoreType.DMA,
        pltpu.SemaphoreType.DMA,
        pltpu.SemaphoreType.DMA,
    ],
)
def _agg_kernel(xs0, xs1, src2d, dst2d, out_hbm,
                gsa, gda, gsb, gdb, rb0, rb1, zblk, acc_sh,
                ga, gb, sa, sb, isem):
    c = lax.axis_index("c")
    s = lax.axis_index("s")
    rows = NP // NS
    base = s * rows
    rbs = (rb0, rb1)
    gsem = (ga, gb)
    ssem = (sa, sb)

    def _zfill(i, _):
        for j in range(DHALF // 16):
            zblk[i, pl.ds(j * 16, 16)] = jnp.zeros((16,), jnp.float32)
        return ()
    lax.fori_loop(0, 16, _zfill, ())

    def _zero(i, _):
        pltpu.sync_copy(zblk, acc_sh.at[pl.ds(base + i * 16, 16), :])
        return ()
    lax.fori_loop(0, rows // 16, _zero, ())
    plsc.subcore_barrier()

    chunk0 = s * _CPT3

    def _run(xs_hbm):
        def g_start(idx_row, b):
            pltpu.async_copy(xs_hbm.at[idx_row], rbs[b], gsem[b])

        def g_wait(b):
            pltpu.make_async_copy(xs_hbm.at[gsa.at[0, 0]], rbs[b],
                                  gsem[b]).wait()

        def s_start(idx_row, b):
            pltpu.async_copy(rbs[b], acc_sh.at[idx_row], ssem[b], add=True)

        def s_wait(b):
            pltpu.make_async_copy(rbs[b], acc_sh.at[gda.at[0, 0]],
                                  ssem[b]).wait()

        def i_wait():
            pltpu.make_async_copy(src2d.at[pl.ds(0, _GRP)], gsa,
                                  isem).wait()

        def do_group(g, cur_src, cur_dst, nxt_src, nxt_dst, first):
            for kk in range(_GRP):
                b = kk % 2
                if not (first and kk < 2):
                    s_wait(b)
                g_start(cur_src.at[kk, 0], b)
                if not (first and kk == 0):
                    g_wait(1 - b)
                    if kk == 0:
                        s_start(nxt_dst.at[_GRP - 1, 0], 1 - b)
                    else:
                        s_start(cur_dst.at[kk - 1, 0], 1 - b)
                if kk == 2:
                    start = chunk0 + jnp.minimum((g + 1) * _GRP,
                                                 _CPT3 - _GRP)
                    pltpu.async_copy(src2d.at[pl.ds(start, _GRP)],
                                     nxt_src, isem)
                    pltpu.async_copy(dst2d.at[pl.ds(start, _GRP)],
                                     nxt_dst, isem)
                if kk == _GRP - 1:
                    i_wait()
                    i_wait()

        pltpu.sync_copy(src2d.at[pl.ds(chunk0, _GRP)], gsa)
        pltpu.sync_copy(dst2d.at[pl.ds(chunk0, _GRP)], gda)
        do_group(0, gsa, gda, gsb, gdb, True)

        def _body(p, _):
            g = 2 * p + 1
            do_group(g, gsb, gdb, gsa, gda, False)
            do_group(g + 1, gsa, gda, gsb, gdb, False)
            return ()
        lax.fori_loop(0, (_NGRP - 2) // 2, _body, ())
        do_group(_NGRP - 1, gsb, gdb, gsa, gda, False)

        g_wait(1)
        s_wait(0)
        s_start(gdb.at[_GRP - 1, 0], 1)
        s_wait(1)

    @pl.when(c == 0)
    def _():
        _run(xs0)

    @pl.when(c == 1)
    def _():
        _run(xs1)

    plsc.subcore_barrier()

    def _out(i, _):
        r0 = base + i * CH
        pltpu.sync_copy(acc_sh.at[pl.ds(r0, CH), :], rb0)
        pltpu.sync_copy(rb0, out_hbm.at[c, pl.ds(r0, CH), :])
        return ()
    lax.fori_loop(0, rows // CH, _out, ())


_DBLK = 512


def _dense_body(dinv_ref, a0_ref, a1_ref, w1_ref, b1_ref, w2_ref, t_ref):
    dinv = dinv_ref[...]
    z0 = (a0_ref[0] * dinv[:, None]).astype(jnp.bfloat16)
    z1 = (a1_ref[0] * dinv[:, None]).astype(jnp.bfloat16)
    h = jnp.dot(z0, w1_ref[0:DHALF, :], preferred_element_type=jnp.float32)
    h += jnp.dot(z1, w1_ref[DHALF:D_IN, :], preferred_element_type=jnp.float32)
    h = jnp.maximum(h + b1_ref[...][None, :], 0.0).astype(jnp.bfloat16)
    s = jnp.dot(h, w2_ref[...], preferred_element_type=jnp.float32)
    t_ref[...] = s[:, 0] * dinv


def _dense(dinv, acc, W1, b1, w2pad):
    return pl.pallas_call(
        _dense_body,
        grid=(NP // _DBLK,),
        in_specs=[
            pl.BlockSpec((_DBLK,), lambda i: (i,)),
            pl.BlockSpec((1, _DBLK, DHALF), lambda i: (0, i, 0)),
            pl.BlockSpec((1, _DBLK, DHALF), lambda i: (1, i, 0)),
            pl.BlockSpec((D_IN, D_H), lambda i: (0, 0)),
            pl.BlockSpec((D_H,), lambda i: (0,)),
            pl.BlockSpec((D_H, 128), lambda i: (0, 0)),
        ],
        out_specs=pl.BlockSpec((_DBLK,), lambda i: (i,)),
        out_shape=jax.ShapeDtypeStruct((NP,), jnp.float32),
    )(dinv, acc, acc, W1, b1, w2pad)


_CPT5 = EP // CH // (NC * NS)


@functools.partial(
    pl.kernel,
    out_type=jax.ShapeDtypeStruct((NC, NP), jnp.float32),
    mesh=_MESH,
    compiler_params=pltpu.CompilerParams(needs_layout_passes=False),
    scratch_types=[
        pltpu.VMEM((_CPT5, 1, CH), jnp.int32),
        pltpu.VMEM((_CPT5, 1, CH), jnp.int32),
        pltpu.VMEM((NP,), jnp.float32),
        pltpu.VMEM((CH,), jnp.float32),
        pltpu.VMEM((CH,), jnp.float32),
        pltpu.VMEM((NP // NS,), jnp.float32),
        pltpu.VMEM_SHARED((NP,), jnp.float32),
        pltpu.SemaphoreType.DMA,
        pltpu.SemaphoreType.DMA,
    ],
)
def _agg2_kernel(t_hbm, src2d, dst2d, out_hbm,
                 gidx, sidx, tloc, msga, msgb, stage_v, acc_sh, sema, semb):
    c = lax.axis_index("c")
    s = lax.axis_index("s")
    rows = NP // NS
    base = s * rows

    def _zero(i, _):
        stage_v[pl.ds(i * 16, 16)] = jnp.zeros((16,), jnp.float32)
        return ()
    lax.fori_loop(0, rows // 16, _zero, ())
    pltpu.sync_copy(stage_v, acc_sh.at[pl.ds(base, rows)])
    chunk0 = (c * NS + s) * _CPT5
    pltpu.sync_copy(src2d.at[pl.ds(chunk0, _CPT5)], gidx)
    pltpu.sync_copy(dst2d.at[pl.ds(chunk0, _CPT5)], sidx)
    pltpu.sync_copy(t_hbm, tloc)
    plsc.subcore_barrier()

    def _msgs(j, buf):
        for v in range(CH // 16):
            idx = gidx[j, 0, pl.ds(v * 16, 16)]
            buf[pl.ds(v * 16, 16)] = plsc.load_gather(tloc, [idx])

    def _s_start(j, buf, sem):
        pltpu.async_copy(buf, acc_sh.at[sidx.at[j, 0]], sem, add=True)

    def _s_wait(buf, sem):
        pltpu.make_async_copy(buf, acc_sh.at[sidx.at[0, 0]], sem).wait()

    _msgs(0, msga)
    _s_start(0, msga, sema)
    _msgs(1, msgb)
    _s_start(1, msgb, semb)

    def _body(p, _):
        j = p * 2
        _s_wait(msga, sema)
        _msgs(j, msga)
        _s_start(j, msga, sema)
        _s_wait(msgb, semb)
        _msgs(j + 1, msgb)
        _s_start(j + 1, msgb, semb)
        return ()
    lax.fori_loop(1, _CPT5 // 2, _body, ())
    _s_wait(msga, sema)
    _s_wait(msgb, semb)
    plsc.subcore_barrier()

    pltpu.sync_copy(acc_sh.at[pl.ds(base, rows)], stage_v)
    pltpu.sync_copy(stage_v, out_hbm.at[c, pl.ds(base, rows)])


def _final_body(part_ref, dinv_ref, b2_ref, out_ref):
    v = (part_ref[0, :] + part_ref[1, :]) * dinv_ref[...] + b2_ref[0]
    out_ref[...] = lax.slice(jax.nn.sigmoid(v), (0,), (N,))


def _finalize(part, dinv, b2):
    return pl.pallas_call(
        _final_body,
        in_specs=[
            pl.BlockSpec((NC, NP), lambda: (0, 0)),
            pl.BlockSpec((NP,), lambda: (0,)),
            pl.BlockSpec(memory_space=pltpu.SMEM),
        ],
        out_specs=pl.BlockSpec((N,), lambda: (0,)),
        out_shape=jax.ShapeDtypeStruct((N,), jnp.float32),
    )(part, dinv, b2)


def kernel(x, edge_index, W1, b1, W2, b2):
    pad_e = EP - E - N
    loop = jnp.arange(N, dtype=jnp.int32)
    trash = (N + (jnp.arange(pad_e, dtype=jnp.int32) % N_TRASH))
    src2d = jnp.concatenate(
        [edge_index[0].astype(jnp.int32), loop, trash]).reshape(EP // CH, 1, CH)
    dst2d = jnp.concatenate(
        [edge_index[1].astype(jnp.int32), loop, trash]).reshape(EP // CH, 1, CH)
    deg2 = _deg_kernel(dst2d)
    dinv, xs0, xs1 = _prescale(deg2.reshape(NC * NP), x)
    acc = _agg_kernel(xs0, xs1, src2d, dst2d)
    w2pad = jnp.pad(W2.astype(jnp.bfloat16), ((0, 0), (0, 127)))
    t = _dense(dinv, acc, W1.astype(jnp.bfloat16), b1, w2pad)
    part = _agg2_kernel(t, src2d, dst2d)
    return _finalize(part, dinv, b2)[:, None]
